# baseline (speedup 1.0000x reference)
import jax
import jax.numpy as jnp
from jax import lax
from jax.experimental import pallas as pl
from jax.experimental.pallas import tpu as pltpu
from jax.experimental.pallas import tpu_sc as plsc

N = 10000
E = 320000
D = 128
H = 128
OUT = 2

NC = 2
NS = 16
NW = NC * NS
NP = 10240
RPT = NP // NS
ZCH = 16
OCH = 128

CH = 128
EPW = E // NW
NCH = EPW // CH
SLAB = 26
NSLAB = NCH // SLAB
TAIL = EPW - NCH * CH

CH2 = 512
NCH2 = E // CH2
BASE_HI = 17 * 20


def _zero_acc(stage, acc, s, width, sem):
    zv = jnp.zeros((16,), jnp.float32)

    def zrow(i, _):
        for g in range(width // 16):
            stage[i, pl.ds(g * 16, 16)] = zv
        return 0
    lax.fori_loop(0, ZCH, zrow, 0)
    hs = []
    for r in range(RPT // ZCH):
        hs.append(pltpu.async_copy(
            stage, acc.at[pl.ds(s * RPT + r * ZCH, ZCH)], sem))
    for h in hs:
        h.wait()


def _copy_out(acc, out, c, s, sem):
    hs = []
    for r in range(RPT // OCH):
        off = s * RPT + r * OCH
        hs.append(pltpu.async_copy(acc.at[pl.ds(off, OCH)],
                                   out.at[c, pl.ds(off, OCH)], sem))
    for h in hs:
        h.wait()


def _make_segsum128():
    mesh = plsc.VectorSubcoreMesh(core_axis_name="c", subcore_axis_name="s")
    out_type = jax.ShapeDtypeStruct((NC, NP, D), jnp.float32)
    scratch = [
        pltpu.VMEM((SLAB * CH,), jnp.int32),
        pltpu.VMEM((SLAB * CH,), jnp.int32),
        pltpu.VMEM((CH, D), jnp.float32),
        pltpu.VMEM((CH, D), jnp.float32),
        pltpu.VMEM((ZCH, D), jnp.float32),
        pltpu.VMEM_SHARED((NP, D), jnp.float32),
        pltpu.SemaphoreType.DMA,
        pltpu.SemaphoreType.DMA,
        pltpu.SemaphoreType.DMA,
        pltpu.SemaphoreType.DMA,
        pltpu.SemaphoreType.DMA,
    ]

    def body(x_hbm, src_hbm, dst_hbm, s_out, srcs, dsts, rows0, rows1,
             stage, acc, sg0, sg1, ss0, ss1, sz):
        c = lax.axis_index("c")
        s = lax.axis_index("s")
        w = c * NS + s
        rows = (rows0, rows1)
        sg = (sg0, sg1)
        ss = (ss0, ss1)

        _zero_acc(stage, acc, s, D, sz)
        plsc.subcore_barrier()

        ebase = w * EPW

        def slab(t, _):
            fbase = pl.multiple_of(ebase + t * (SLAB * CH), 8)
            pltpu.sync_copy(src_hbm.at[pl.ds(fbase, SLAB * CH)], srcs)
            pltpu.sync_copy(dst_hbm.at[pl.ds(fbase, SLAB * CH)], dsts)
            gd = [None, None]
            sd = [None, None]
            gd[0] = pltpu.async_copy(
                x_hbm.at[srcs.at[pl.ds(0, CH)]], rows0, sg0)
            for b in range(SLAB):
                cur = b % 2
                nxt = 1 - cur
                if b + 1 < SLAB:
                    if sd[nxt] is not None:
                        sd[nxt].wait()
                    gd[nxt] = pltpu.async_copy(
                        x_hbm.at[srcs.at[pl.ds((b + 1) * CH, CH)]],
                        rows[nxt], sg[nxt])
                gd[cur].wait()
                sd[cur] = pltpu.async_copy(
                    rows[cur], acc.at[dsts.at[pl.ds(b * CH, CH)]],
                    ss[cur], add=True)
            for slot in range(2):
                if sd[slot] is not None:
                    sd[slot].wait()
            return 0
        lax.fori_loop(0, NSLAB, slab, 0)

        tbase = pl.multiple_of(ebase + NCH * CH, 8)
        pltpu.sync_copy(src_hbm.at[pl.ds(tbase, TAIL)],
                        srcs.at[pl.ds(0, TAIL)])
        pltpu.sync_copy(dst_hbm.at[pl.ds(tbase, TAIL)],
                        dsts.at[pl.ds(0, TAIL)])
        pltpu.async_copy(x_hbm.at[srcs.at[pl.ds(0, TAIL)]],
                         rows0.at[pl.ds(0, TAIL)], sg0).wait()
        pltpu.sync_copy(rows0.at[pl.ds(0, TAIL)],
                        acc.at[dsts.at[pl.ds(0, TAIL)]], add=True)

        plsc.subcore_barrier()
        _copy_out(acc, s_out, c, s, sz)

    params = pltpu.CompilerParams(use_tc_tiling_on_sc=False)
    return pl.kernel(body, out_type=out_type, mesh=mesh,
                     scratch_types=scratch, compiler_params=params)


def _w2_chunk_base(w):
    return jnp.where(w < 17, 20 * w, BASE_HI + 19 * (w - 17))


def _rsqrt_nr(s2):
    y = plsc.bitcast(s2, jnp.int32)
    y = jnp.int32(0x5F3759DF) - lax.shift_right_logical(y, 1)
    f = plsc.bitcast(y, jnp.float32)
    for _ in range(3):
        f = f * (1.5 - 0.5 * s2 * f * f)
    return f


def _log1p_small(z):
    w = z / (2.0 + z)
    w2 = w * w
    return 2.0 * w * (1.0 + w2 * (1.0 / 3.0 + w2 * (0.2 + w2 / 7.0)))


def _make_segsum16_final():
    NSLOT = 4
    CH3 = 512
    S16 = 13
    CPT = 39
    mesh = plsc.VectorSubcoreMesh(core_axis_name="c", subcore_axis_name="s")
    out_type = jax.ShapeDtypeStruct((NP, 16), jnp.float32)
    scratch = [
        pltpu.VMEM((S16 * CH3,), jnp.int32),
        pltpu.VMEM((S16 * CH3,), jnp.int32),
        [pltpu.VMEM((CH3, 16), jnp.float32) for _ in range(NSLOT)],
        pltpu.VMEM((ZCH, 16), jnp.float32),
        pltpu.VMEM((NP // NW, 16), jnp.float32),
        pltpu.VMEM((NP // NW, 16), jnp.float32),
        pltpu.VMEM((NP // NW, 16), jnp.float32),
        pltpu.VMEM((NP // NW, 16), jnp.float32),
        pltpu.VMEM((NP // NW, 16), jnp.float32),
        pltpu.VMEM_SHARED((NP, 16), jnp.float32),
        pltpu.VMEM_SHARED((NP, 16), jnp.float32),
        [pltpu.SemaphoreType.DMA for _ in range(NSLOT)],
        [pltpu.SemaphoreType.DMA for _ in range(NSLOT)],
        pltpu.SemaphoreType.DMA,
    ]
    TPT = N // NS

    def body(x_hbm, src_hbm, dst_hbm, cnt_hbm, out_hbm, srcs, dsts, rows,
             stage, s2b, p2b, cb0, cb1, outb, tbl, acc, sg, ss, sz):
        c = lax.axis_index("c")
        s = lax.axis_index("s")

        ii0 = jnp.arange(16, dtype=jnp.int32)
        zv16 = jnp.zeros((16,), jnp.float32)
        for i in range(ZCH):
            plsc.store_scatter(stage, [jnp.full((16,), i, jnp.int32), ii0],
                               zv16)
        hs = []
        for r in range(RPT // ZCH):
            hs.append(pltpu.async_copy(
                stage, acc.at[pl.ds(s * RPT + r * ZCH, ZCH)], sz))
        for h in hs:
            h.wait()
        pltpu.async_copy(x_hbm.at[pl.ds(s * TPT, TPT)],
                         tbl.at[pl.ds(s * TPT, TPT)], sz).wait()
        plsc.subcore_barrier()

        cbase = CPT * s
        LA = NSLOT - 1

        def slab(t, _):
            ebase = pl.multiple_of((cbase + t * S16) * CH3, 8)
            pltpu.sync_copy(src_hbm.at[pl.ds(ebase, S16 * CH3)], srcs)
            pltpu.sync_copy(dst_hbm.at[pl.ds(ebase, S16 * CH3)], dsts)
            gd = [None] * NSLOT
            sd = [None] * NSLOT

            def gop(b, slot):
                return pltpu.async_copy(
                    tbl.at[srcs.at[pl.ds(b * CH3, CH3)]], rows[slot],
                    sg[slot])

            for k in range(LA):
                gd[k] = gop(k, k)
            for b in range(S16):
                cur = b % NSLOT
                nb = b + LA
                if nb < S16:
                    slot = nb % NSLOT
                    if sd[slot] is not None:
                        sd[slot].wait()
                    gd[slot] = gop(nb, slot)
                gd[cur].wait()
                sd[cur] = pltpu.async_copy(
                    rows[cur], acc.at[dsts.at[pl.ds(b * CH3, CH3)]],
                    ss[cur], add=True)
            for slot in range(NSLOT):
                if sd[slot] is not None:
                    sd[slot].wait()
            return 0
        lax.fori_loop(0, CPT // S16, slab, 0)

        @pl.when(s == 15)
        def _():
            ebase = pl.multiple_of(624 * CH3, 8)
            pltpu.sync_copy(src_hbm.at[pl.ds(ebase, CH3)],
                            srcs.at[pl.ds(0, CH3)])
            pltpu.sync_copy(dst_hbm.at[pl.ds(ebase, CH3)],
                            dsts.at[pl.ds(0, CH3)])
            pltpu.async_copy(tbl.at[srcs.at[pl.ds(0, CH3)]], rows[0],
                             sg[0]).wait()
            pltpu.sync_copy(rows[0], acc.at[dsts.at[pl.ds(0, CH3)]],
                            add=True)

        plsc.subcore_barrier()

        FPW = NP // NW
        nbase = (c * NS + s) * FPW
        pltpu.sync_copy(acc.at[pl.ds(nbase, FPW)], s2b.at[pl.ds(0, FPW)])
        pltpu.sync_copy(tbl.at[pl.ds(nbase, FPW)], p2b.at[pl.ds(0, FPW)])
        pltpu.sync_copy(cnt_hbm.at[0, pl.ds(nbase, FPW)],
                        cb0.at[pl.ds(0, FPW)])
        pltpu.sync_copy(cnt_hbm.at[1, pl.ds(nbase, FPW)],
                        cb1.at[pl.ds(0, FPW)])

        ii = jnp.arange(16, dtype=jnp.int32)
        for g in range(FPW // 16):
            ridx = ii + g * 16
            col = [jnp.full((16,), j, jnp.int32) for j in range(5)]
            p0 = plsc.load_gather(s2b, [ridx, col[0]])
            p1 = plsc.load_gather(s2b, [ridx, col[1]])
            r0 = plsc.load_gather(p2b, [ridx, col[2]])
            r1 = plsc.load_gather(p2b, [ridx, col[3]])
            cnt = (plsc.load_gather(cb0, [ridx, col[0]])
                   + plsc.load_gather(cb1, [ridx, col[0]]))
            rcpc = 1.0 / jnp.maximum(cnt, 1.0)
            v0 = p0 * rcpc + r0
            v1 = p1 * rcpc + r1
            s2 = jnp.maximum(v0 * v0 + v1 * v1, 1e-24)
            rs = _rsqrt_nr(s2)
            v0 = v0 * rs
            v1 = v1 * rs
            m = jnp.maximum(v0, v1)
            d0 = v0 - m
            d1 = v1 - m
            z = jnp.exp(jnp.minimum(d0, d1))
            ls = _log1p_small(z)
            plsc.store_scatter(outb, [ridx, col[0]], d0 - ls)
            plsc.store_scatter(outb, [ridx, col[1]], d1 - ls)
        pltpu.sync_copy(outb.at[pl.ds(0, FPW)],
                        out_hbm.at[pl.ds(nbase, FPW)])

    params = pltpu.CompilerParams(use_tc_tiling_on_sc=False,
                                  needs_layout_passes=False)
    return pl.kernel(body, out_type=out_type, mesh=mesh,
                     scratch_types=scratch, compiler_params=params)


def _make_counts():
    mesh = plsc.VectorSubcoreMesh(core_axis_name="c", subcore_axis_name="s")
    out_type = jax.ShapeDtypeStruct((NC, NP, 16), jnp.float32)
    scratch = [
        pltpu.VMEM((20 * CH2,), jnp.int32),
        pltpu.VMEM((CH2, 16), jnp.float32),
        pltpu.VMEM((ZCH, 16), jnp.float32),
        pltpu.VMEM_SHARED((NP, 16), jnp.float32),
        pltpu.SemaphoreType.DMA,
        pltpu.SemaphoreType.DMA,
    ]

    def body(dst_hbm, c_out, dsts, ones, stage, cacc, sc, sz):
        c = lax.axis_index("c")
        s = lax.axis_index("s")
        w = c * NS + s
        ov = jnp.ones((16,), jnp.float32)

        def orow(i, _):
            ones[i, :] = ov
            return 0
        lax.fori_loop(0, CH2, orow, 0)
        _zero_acc(stage, cacc, s, 16, sz)

        ebase = pl.multiple_of(_w2_chunk_base(w) * CH2, 8)
        pltpu.sync_copy(dst_hbm.at[pl.ds(ebase, 19 * CH2)],
                        dsts.at[pl.ds(0, 19 * CH2)])

        @pl.when(w < 17)
        def _():
            xb = pl.multiple_of(ebase + 19 * CH2, 8)
            pltpu.sync_copy(dst_hbm.at[pl.ds(xb, CH2)],
                            dsts.at[pl.ds(19 * CH2, CH2)])
        plsc.subcore_barrier()

        sd = []
        for b in range(19):
            sd.append(pltpu.async_copy(
                ones, cacc.at[dsts.at[pl.ds(b * CH2, CH2)]], sc, add=True))
        for h in sd:
            h.wait()

        @pl.when(w < 17)
        def _():
            pltpu.sync_copy(ones, cacc.at[dsts.at[pl.ds(19 * CH2, CH2)]],
                            add=True)

        plsc.subcore_barrier()
        _copy_out(cacc, c_out, c, s, sz)

    params = pltpu.CompilerParams(use_tc_tiling_on_sc=False)
    return pl.kernel(body, out_type=out_type, mesh=mesh,
                     scratch_types=scratch, compiler_params=params)


_segsum128 = _make_segsum128()
_segsum16f = _make_segsum16_final()
_counts = _make_counts()

BN = 2000


def _tc1_body(s_ref, c_ref, x_ref, w1l_ref, w1r_ref, b1_ref, w2_ref,
              b2_ref, p_ref):
    S = s_ref[0] + s_ref[1]
    cnt = c_ref[0, :, 0:1] + c_ref[1, :, 0:1]
    agg = S / jnp.maximum(cnt, 1.0)
    t = jnp.dot(agg, w1l_ref[...], preferred_element_type=jnp.float32)
    t = t + jnp.dot(x_ref[...], w1r_ref[...], preferred_element_type=jnp.float32)
    t = t + b1_ref[...]
    nrm = jnp.sqrt(jnp.sum(t * t, axis=1, keepdims=True))
    h = jnp.maximum(t / jnp.maximum(nrm, 1e-12), 0.0)
    p_ref[...] = (jnp.dot(h, w2_ref[...], preferred_element_type=jnp.float32)
                  + b2_ref[...])


def _tc1(S1p, C1p, x, W1_l, W1_r, b1r, W2cat, b2p):
    return pl.pallas_call(
        _tc1_body,
        grid=(N // BN,),
        in_specs=[
            pl.BlockSpec((NC, BN, D), lambda i: (0, i, 0)),
            pl.BlockSpec((NC, BN, 16), lambda i: (0, i, 0)),
            pl.BlockSpec((BN, D), lambda i: (i, 0)),
            pl.BlockSpec((D, H), lambda i: (0, 0)),
            pl.BlockSpec((D, H), lambda i: (0, 0)),
            pl.BlockSpec((1, H), lambda i: (0, 0)),
            pl.BlockSpec((H, 16), lambda i: (0, 0)),
            pl.BlockSpec((1, 16), lambda i: (0, 0)),
        ],
        out_specs=pl.BlockSpec((BN, 16), lambda i: (i, 0)),
        out_shape=jax.ShapeDtypeStruct((N, 16), jnp.float32),
    )(S1p, C1p, x, W1_l, W1_r, b1r, W2cat, b2p)


def kernel(x, edge_index, W1_l, W1_r, b1, W2_l, W2_r, b2):
    src = edge_index[0]
    dst = edge_index[1]
    S1p = _segsum128(x, src, dst)
    C1p = _counts(dst)
    W2cat = (jnp.zeros((H, 16), jnp.float32)
             .at[:, 0:2].set(W2_l).at[:, 2:4].set(W2_r))
    b2p = jnp.zeros((1, 16), jnp.float32).at[0, 2:4].set(b2)
    p2s = _tc1(S1p, C1p, x, W1_l, W1_r, b1.reshape(1, H), W2cat, b2p)
    out_full = _segsum16f(p2s, src, dst, C1p)
    return out_full[:N, :OUT]


# speedup vs baseline: 1.0720x; 1.0720x over previous
import jax
import jax.numpy as jnp
from jax import lax
from jax.experimental import pallas as pl
from jax.experimental.pallas import tpu as pltpu
from jax.experimental.pallas import tpu_sc as plsc

N = 10000
E = 320000
D = 128
H = 128
OUT = 2

NC = 2
NS = 16
NW = NC * NS
NP = 10240
RPT = NP // NS
ZCH = 16
OCH = 128

CH = 128
EPW = E // NW
NCH = EPW // CH
SLAB = 26
NSLAB = NCH // SLAB
TAIL = EPW - NCH * CH

CH2 = 512
NCH2 = E // CH2
BASE_HI = 17 * 20


def _zero_acc(stage, acc, s, width, sem):
    zv = jnp.zeros((16,), jnp.float32)

    def zrow(i, _):
        for g in range(width // 16):
            stage[i, pl.ds(g * 16, 16)] = zv
        return 0
    lax.fori_loop(0, ZCH, zrow, 0)
    hs = []
    for r in range(RPT // ZCH):
        hs.append(pltpu.async_copy(
            stage, acc.at[pl.ds(s * RPT + r * ZCH, ZCH)], sem))
    for h in hs:
        h.wait()


def _copy_out(acc, out, c, s, sem):
    hs = []
    for r in range(RPT // OCH):
        off = s * RPT + r * OCH
        hs.append(pltpu.async_copy(acc.at[pl.ds(off, OCH)],
                                   out.at[c, pl.ds(off, OCH)], sem))
    for h in hs:
        h.wait()


def _make_segsum128():
    mesh = plsc.VectorSubcoreMesh(core_axis_name="c", subcore_axis_name="s")
    out_type = jax.ShapeDtypeStruct((NC, NP, D), jnp.float32)
    scratch = [
        pltpu.VMEM((SLAB * CH,), jnp.int32),
        pltpu.VMEM((SLAB * CH,), jnp.int32),
        pltpu.VMEM((CH, D), jnp.float32),
        pltpu.VMEM((CH, D), jnp.float32),
        pltpu.VMEM((ZCH, D), jnp.float32),
        pltpu.VMEM_SHARED((NP, D), jnp.float32),
        pltpu.SemaphoreType.DMA,
        pltpu.SemaphoreType.DMA,
        pltpu.SemaphoreType.DMA,
        pltpu.SemaphoreType.DMA,
        pltpu.SemaphoreType.DMA,
    ]

    def body(x_hbm, ei_hbm, s_out, srcs, dsts, rows0, rows1,
             stage, acc, sg0, sg1, ss0, ss1, sz):
        c = lax.axis_index("c")
        s = lax.axis_index("s")
        w = c * NS + s
        rows = (rows0, rows1)
        sg = (sg0, sg1)
        ss = (ss0, ss1)

        _zero_acc(stage, acc, s, D, sz)
        plsc.subcore_barrier()

        ebase = w * EPW

        def slab(t, _):
            fbase = pl.multiple_of(ebase + t * (SLAB * CH), 8)
            pltpu.sync_copy(ei_hbm.at[0, pl.ds(fbase, SLAB * CH)], srcs)
            pltpu.sync_copy(ei_hbm.at[1, pl.ds(fbase, SLAB * CH)], dsts)
            gd = [None, None]
            sd = [None, None]
            gd[0] = pltpu.async_copy(
                x_hbm.at[srcs.at[pl.ds(0, CH)]], rows0, sg0)
            for b in range(SLAB):
                cur = b % 2
                nxt = 1 - cur
                if b + 1 < SLAB:
                    if sd[nxt] is not None:
                        sd[nxt].wait()
                    gd[nxt] = pltpu.async_copy(
                        x_hbm.at[srcs.at[pl.ds((b + 1) * CH, CH)]],
                        rows[nxt], sg[nxt])
                gd[cur].wait()
                sd[cur] = pltpu.async_copy(
                    rows[cur], acc.at[dsts.at[pl.ds(b * CH, CH)]],
                    ss[cur], add=True)
            for slot in range(2):
                if sd[slot] is not None:
                    sd[slot].wait()
            return 0
        lax.fori_loop(0, NSLAB, slab, 0)

        tbase = pl.multiple_of(ebase + NCH * CH, 8)
        pltpu.sync_copy(ei_hbm.at[0, pl.ds(tbase, TAIL)],
                        srcs.at[pl.ds(0, TAIL)])
        pltpu.sync_copy(ei_hbm.at[1, pl.ds(tbase, TAIL)],
                        dsts.at[pl.ds(0, TAIL)])
        pltpu.async_copy(x_hbm.at[srcs.at[pl.ds(0, TAIL)]],
                         rows0.at[pl.ds(0, TAIL)], sg0).wait()
        pltpu.sync_copy(rows0.at[pl.ds(0, TAIL)],
                        acc.at[dsts.at[pl.ds(0, TAIL)]], add=True)

        plsc.subcore_barrier()
        _copy_out(acc, s_out, c, s, sz)

    params = pltpu.CompilerParams(use_tc_tiling_on_sc=False)
    return pl.kernel(body, out_type=out_type, mesh=mesh,
                     scratch_types=scratch, compiler_params=params)


def _w2_chunk_base(w):
    return jnp.where(w < 17, 20 * w, BASE_HI + 19 * (w - 17))


def _make_segsum16():
    NSLOT = 4
    mesh = plsc.VectorSubcoreMesh(core_axis_name="c", subcore_axis_name="s")
    out_type = jax.ShapeDtypeStruct((NC, NP, 16), jnp.float32)
    scratch = [
        pltpu.VMEM((20 * CH2,), jnp.int32),
        pltpu.VMEM((20 * CH2,), jnp.int32),
        [pltpu.VMEM((CH2, 16), jnp.float32) for _ in range(NSLOT)],
        pltpu.VMEM((ZCH, 16), jnp.float32),
        pltpu.VMEM_SHARED((N, 16), jnp.float32),
        pltpu.VMEM_SHARED((NP, 16), jnp.float32),
        [pltpu.SemaphoreType.DMA for _ in range(NSLOT)],
        [pltpu.SemaphoreType.DMA for _ in range(NSLOT)],
        pltpu.SemaphoreType.DMA,
    ]
    TPT = N // NS

    def body(x_hbm, ei_hbm, s_out, srcs, dsts, rows, stage,
             tbl, acc, sg, ss, sz):
        c = lax.axis_index("c")
        s = lax.axis_index("s")
        w = c * NS + s

        _zero_acc(stage, acc, s, 16, sz)
        pltpu.async_copy(x_hbm.at[pl.ds(s * TPT, TPT)],
                         tbl.at[pl.ds(s * TPT, TPT)], sz).wait()
        ebase = pl.multiple_of(_w2_chunk_base(w) * CH2, 8)
        pltpu.sync_copy(ei_hbm.at[0, pl.ds(ebase, 19 * CH2)],
                        srcs.at[pl.ds(0, 19 * CH2)])
        pltpu.sync_copy(ei_hbm.at[1, pl.ds(ebase, 19 * CH2)],
                        dsts.at[pl.ds(0, 19 * CH2)])

        @pl.when(w < 17)
        def _():
            xb = pl.multiple_of(ebase + 19 * CH2, 8)
            pltpu.sync_copy(ei_hbm.at[0, pl.ds(xb, CH2)],
                            srcs.at[pl.ds(19 * CH2, CH2)])
            pltpu.sync_copy(ei_hbm.at[1, pl.ds(xb, CH2)],
                            dsts.at[pl.ds(19 * CH2, CH2)])
        plsc.subcore_barrier()

        LA = NSLOT - 1
        gd = [None] * NSLOT
        sd = [None] * NSLOT

        def gop(b, slot):
            return pltpu.async_copy(
                tbl.at[srcs.at[pl.ds(b * CH2, CH2)]], rows[slot], sg[slot])

        for k in range(LA):
            gd[k] = gop(k, k)
        for b in range(19):
            cur = b % NSLOT
            nb = b + LA
            if nb < 19:
                slot = nb % NSLOT
                if sd[slot] is not None:
                    sd[slot].wait()
                gd[slot] = gop(nb, slot)
            gd[cur].wait()
            sd[cur] = pltpu.async_copy(
                rows[cur], acc.at[dsts.at[pl.ds(b * CH2, CH2)]],
                ss[cur], add=True)
        for slot in range(NSLOT):
            if sd[slot] is not None:
                sd[slot].wait()

        @pl.when(w < 17)
        def _():
            gop(19, 0).wait()
            pltpu.sync_copy(rows[0], acc.at[dsts.at[pl.ds(19 * CH2, CH2)]],
                            add=True)

        plsc.subcore_barrier()
        _copy_out(acc, s_out, c, s, sz)

    params = pltpu.CompilerParams(use_tc_tiling_on_sc=False)
    return pl.kernel(body, out_type=out_type, mesh=mesh,
                     scratch_types=scratch, compiler_params=params)


def _make_counts():
    mesh = plsc.VectorSubcoreMesh(core_axis_name="c", subcore_axis_name="s")
    out_type = jax.ShapeDtypeStruct((NC, NP, 16), jnp.float32)
    scratch = [
        pltpu.VMEM((20 * CH2,), jnp.int32),
        pltpu.VMEM((CH2, 16), jnp.float32),
        pltpu.VMEM((ZCH, 16), jnp.float32),
        pltpu.VMEM_SHARED((NP, 16), jnp.float32),
        pltpu.SemaphoreType.DMA,
        pltpu.SemaphoreType.DMA,
    ]

    def body(ei_hbm, c_out, dsts, ones, stage, cacc, sc, sz):
        c = lax.axis_index("c")
        s = lax.axis_index("s")
        w = c * NS + s
        ov = jnp.ones((16,), jnp.float32)

        def orow(i, _):
            ones[i, :] = ov
            return 0
        lax.fori_loop(0, CH2, orow, 0)
        _zero_acc(stage, cacc, s, 16, sz)

        ebase = pl.multiple_of(_w2_chunk_base(w) * CH2, 8)
        pltpu.sync_copy(ei_hbm.at[1, pl.ds(ebase, 19 * CH2)],
                        dsts.at[pl.ds(0, 19 * CH2)])

        @pl.when(w < 17)
        def _():
            xb = pl.multiple_of(ebase + 19 * CH2, 8)
            pltpu.sync_copy(ei_hbm.at[1, pl.ds(xb, CH2)],
                            dsts.at[pl.ds(19 * CH2, CH2)])
        plsc.subcore_barrier()

        sd = []
        for b in range(19):
            sd.append(pltpu.async_copy(
                ones, cacc.at[dsts.at[pl.ds(b * CH2, CH2)]], sc, add=True))
        for h in sd:
            h.wait()

        @pl.when(w < 17)
        def _():
            pltpu.sync_copy(ones, cacc.at[dsts.at[pl.ds(19 * CH2, CH2)]],
                            add=True)

        plsc.subcore_barrier()
        _copy_out(cacc, c_out, c, s, sz)

    params = pltpu.CompilerParams(use_tc_tiling_on_sc=False)
    return pl.kernel(body, out_type=out_type, mesh=mesh,
                     scratch_types=scratch, compiler_params=params)


_segsum128 = _make_segsum128()
_segsum16 = _make_segsum16()
_counts = _make_counts()

BN = 2000


def _tc1_body(s_ref, c_ref, x_ref, w1l_ref, w1r_ref, b1_ref, w2_ref, p_ref):
    S = s_ref[0] + s_ref[1]
    cnt = c_ref[0, :, 0:1] + c_ref[1, :, 0:1]
    agg = S / jnp.maximum(cnt, 1.0)
    t = jnp.dot(agg, w1l_ref[...], preferred_element_type=jnp.float32)
    t = t + jnp.dot(x_ref[...], w1r_ref[...], preferred_element_type=jnp.float32)
    t = t + b1_ref[...]
    nrm = jnp.sqrt(jnp.sum(t * t, axis=1, keepdims=True))
    h = jnp.maximum(t / jnp.maximum(nrm, 1e-12), 0.0)
    p_ref[...] = jnp.dot(h, w2_ref[...], preferred_element_type=jnp.float32)


def _tc1(S1p, C1p, x, W1_l, W1_r, b1r, W2cat):
    return pl.pallas_call(
        _tc1_body,
        grid=(N // BN,),
        in_specs=[
            pl.BlockSpec((NC, BN, D), lambda i: (0, i, 0)),
            pl.BlockSpec((NC, BN, 16), lambda i: (0, i, 0)),
            pl.BlockSpec((BN, D), lambda i: (i, 0)),
            pl.BlockSpec((D, H), lambda i: (0, 0)),
            pl.BlockSpec((D, H), lambda i: (0, 0)),
            pl.BlockSpec((1, H), lambda i: (0, 0)),
            pl.BlockSpec((H, 16), lambda i: (0, 0)),
        ],
        out_specs=pl.BlockSpec((BN, 16), lambda i: (i, 0)),
        out_shape=jax.ShapeDtypeStruct((N, 16), jnp.float32),
    )(S1p, C1p, x, W1_l, W1_r, b1r, W2cat)


def _tc2_body(s2_ref, c_ref, p_ref, b2_ref, o_ref):
    S2 = s2_ref[0] + s2_ref[1]
    cnt = c_ref[0, :, 0:1] + c_ref[1, :, 0:1]
    agg = S2[:, 0:2] / jnp.maximum(cnt, 1.0)
    v = agg + p_ref[:, 2:4] + b2_ref[...]
    nrm = jnp.sqrt(jnp.sum(v * v, axis=1, keepdims=True))
    v = v / jnp.maximum(nrm, 1e-12)
    m = jnp.max(v, axis=1, keepdims=True)
    e = jnp.exp(v - m)
    o_ref[...] = (v - m) - jnp.log(jnp.sum(e, axis=1, keepdims=True))


def _tc2(S2p, C1p, p2s, b2r):
    return pl.pallas_call(
        _tc2_body,
        grid=(N // BN,),
        in_specs=[
            pl.BlockSpec((NC, BN, 16), lambda i: (0, i, 0)),
            pl.BlockSpec((NC, BN, 16), lambda i: (0, i, 0)),
            pl.BlockSpec((BN, 16), lambda i: (i, 0)),
            pl.BlockSpec((1, OUT), lambda i: (0, 0)),
        ],
        out_specs=pl.BlockSpec((BN, OUT), lambda i: (i, 0)),
        out_shape=jax.ShapeDtypeStruct((N, OUT), jnp.float32),
    )(S2p, C1p, p2s, b2r)


def kernel(x, edge_index, W1_l, W1_r, b1, W2_l, W2_r, b2):
    S1p = _segsum128(x, edge_index)
    C1p = _counts(edge_index)
    W2cat = (jnp.zeros((H, 16), jnp.float32)
             .at[:, 0:2].set(W2_l).at[:, 2:4].set(W2_r))
    p2s = _tc1(S1p, C1p, x, W1_l, W1_r, b1.reshape(1, H), W2cat)
    S2p = _segsum16(p2s, edge_index)
    out = _tc2(S2p, C1p, p2s, b2.reshape(1, OUT))
    return out
